# baseline (speedup 1.0000x reference)
import functools

import jax
import jax.numpy as jnp
from jax import lax
from jax.experimental import pallas as pl
from jax.experimental.pallas import tpu as pltpu
from jax.experimental.pallas import tpu_sc as plsc

NUM_CLASSES = 1000
NUM_ROWS = 1024 * 50
NC, NS, L = 2, 16, 16
NW = NC * NS
ROWS_PER_W = NUM_ROWS // NW
CHUNK = 32
NCHUNK = ROWS_PER_W // CHUNK
CHUNK_WORDS = CHUNK * NUM_CLASSES


def _body(idx_hbm, out_hbm, buf0, buf1, idx_v, sem0, sem1):
    wid = lax.axis_index("s") * NC + lax.axis_index("c")
    base_row = wid * ROWS_PER_W

    pltpu.sync_copy(idx_hbm.at[pl.ds(base_row, ROWS_PER_W)], idx_v)

    zeros = jnp.zeros((L,), jnp.float32)
    ones = jnp.ones((L,), jnp.float32)
    lane = lax.iota(jnp.int32, L)

    def zero_step(i, carry):
        buf0[pl.ds(i * L, L)] = zeros
        buf1[pl.ds(i * L, L)] = zeros
        return carry

    lax.fori_loop(0, CHUNK_WORDS // L, zero_step, 0)

    def scatter(buf, c, vec):
        for g in range(CHUNK // L):
            vals = idx_v[pl.ds(c * CHUNK + g * L, L)]
            pos = (lane + g * L) * NUM_CLASSES + vals
            plsc.store_scatter(buf, [pos], vec)

    bufs = (buf0, buf1)
    sems = (sem0, sem1)
    copies = [None, None]
    for c in range(NCHUNK):
        b = c % 2
        if copies[b] is not None:
            copies[b].wait()
            scatter(bufs[b], c - 2, zeros)
        scatter(bufs[b], c, ones)
        off = (base_row + c * CHUNK) * NUM_CLASSES
        cp = pltpu.make_async_copy(
            bufs[b], out_hbm.at[pl.ds(off, CHUNK_WORDS)], sems[b]
        )
        cp.start()
        copies[b] = cp
    copies[0].wait()
    copies[1].wait()


@jax.jit
def _one_hot(idx):
    mesh = plsc.VectorSubcoreMesh(core_axis_name="c", subcore_axis_name="s")
    run = pl.kernel(
        _body,
        out_type=jax.ShapeDtypeStruct((NUM_ROWS * NUM_CLASSES,), jnp.float32),
        mesh=mesh,
        compiler_params=pltpu.CompilerParams(needs_layout_passes=False),
        scratch_types=[
            pltpu.VMEM((CHUNK_WORDS,), jnp.float32),
            pltpu.VMEM((CHUNK_WORDS,), jnp.float32),
            pltpu.VMEM((ROWS_PER_W,), jnp.int32),
            pltpu.SemaphoreType.DMA,
            pltpu.SemaphoreType.DMA,
        ],
    )
    return run(idx)


def kernel(inputs):
    idx = inputs.reshape(-1).astype(jnp.int32)
    out = _one_hot(idx)
    return out.reshape(inputs.shape[0], inputs.shape[1], NUM_CLASSES)


# speedup vs baseline: 1.8362x; 1.8362x over previous
import jax
import jax.numpy as jnp
from jax import lax
from jax.experimental import pallas as pl
from jax.experimental.pallas import tpu as pltpu
from jax.experimental.pallas import tpu_sc as plsc

BATCH = 1024
ROWS = 50
NUM_CLASSES = 1000
NC, NS, L = 2, 16, 16
NW = NC * NS
B_PER_W = BATCH // NW
SLAB_WORDS = ROWS * NUM_CLASSES


def _body(idx_hbm, out_hbm, buf0, buf1, idx_v, sem0, sem1):
    wid = lax.axis_index("s") * NC + lax.axis_index("c")
    base_b = wid * B_PER_W

    pltpu.sync_copy(
        idx_hbm.at[pl.ds(base_b * ROWS, B_PER_W * ROWS)],
        idx_v.at[pl.ds(0, B_PER_W * ROWS)],
    )

    zeros = jnp.zeros((L,), jnp.float32)
    ones = jnp.ones((L,), jnp.float32)
    lane = lax.iota(jnp.int32, L)

    def zero_step(i, carry):
        r = i // (NUM_CLASSES // L)
        k = (i % (NUM_CLASSES // L)) * L
        buf0[r, pl.ds(k, L)] = zeros
        buf1[r, pl.ds(k, L)] = zeros
        return carry

    lax.fori_loop(0, SLAB_WORDS // L, zero_step, 0)

    def scatter(buf, i, vec):
        for g in range(-(-ROWS // L)):
            vals = idx_v[pl.ds(i * ROWS + g * L, L)]
            rows = lane + g * L
            cnt = min(L, ROWS - g * L)
            mask = None if cnt == L else lane < cnt
            plsc.store_scatter(buf, [rows, vals], vec, mask=mask)

    bufs = (buf0, buf1)
    sems = (sem0, sem1)
    copies = [None, None]
    for i in range(B_PER_W):
        b = i % 2
        if copies[b] is not None:
            copies[b].wait()
            scatter(bufs[b], i - 2, zeros)
        scatter(bufs[b], i, ones)
        cp = pltpu.make_async_copy(bufs[b], out_hbm.at[base_b + i], sems[b])
        cp.start()
        copies[b] = cp
    copies[0].wait()
    copies[1].wait()


@jax.jit
def _one_hot(idx):
    mesh = plsc.VectorSubcoreMesh(core_axis_name="c", subcore_axis_name="s")
    run = pl.kernel(
        _body,
        out_type=jax.ShapeDtypeStruct((BATCH, ROWS, NUM_CLASSES), jnp.float32),
        mesh=mesh,
        compiler_params=pltpu.CompilerParams(
            needs_layout_passes=False, use_tc_tiling_on_sc=True
        ),
        scratch_types=[
            pltpu.VMEM((ROWS, NUM_CLASSES), jnp.float32),
            pltpu.VMEM((ROWS, NUM_CLASSES), jnp.float32),
            pltpu.VMEM((B_PER_W * ROWS + L,), jnp.int32),
            pltpu.SemaphoreType.DMA,
            pltpu.SemaphoreType.DMA,
        ],
    )
    return run(idx)


def kernel(inputs):
    idx = inputs.reshape(-1).astype(jnp.int32)
    return _one_hot(idx)


# speedup vs baseline: 5.2334x; 2.8501x over previous
import jax
import jax.numpy as jnp
from jax import lax
from jax.experimental import pallas as pl
from jax.experimental.pallas import tpu as pltpu
from jax.experimental.pallas import tpu_sc as plsc

BATCH = 1024
ROWS = 50
NUM_CLASSES = 1000
NC, NS, L = 2, 16, 16
NW = NC * NS
KCH = 40
NKC = NUM_CLASSES // KCH
UNITS = ROWS * NKC
GROUPS = BATCH // L
NR_PRE = 3


def _body(idx_hbm, out_hbm, buf0, buf1, idx_v, sem0, sem1):
    wid = lax.axis_index("s") * NC + lax.axis_index("c")
    u0 = wid * UNITS // NW
    u1 = (wid + 1) * UNITS // NW
    n = u1 - u0
    base_r = u0 // NKC

    pltpu.sync_copy(idx_hbm.at[pl.ds(base_r * BATCH, NR_PRE * BATCH)], idx_v)

    zeros = jnp.zeros((L,), jnp.float32)
    ones = jnp.ones((L,), jnp.float32)
    lane = lax.iota(jnp.int32, L)

    def zero_step(i, carry):
        r = i // (BATCH // L)
        c = (i % (BATCH // L)) * L
        buf0[r, pl.ds(c, L)] = zeros
        buf1[r, pl.ds(c, L)] = zeros
        return carry

    lax.fori_loop(0, KCH * (BATCH // L), zero_step, 0)

    def scatter(buf, u, vec):
        r_off = u // NKC - base_r
        k0 = (u % NKC) * KCH
        for g in range(GROUPS):
            vals = idx_v[pl.ds(r_off * BATCH + g * L, L)]
            lk = vals - k0
            in_win = (lk >= 0) & (lk < KCH)
            lk = jnp.where(in_win, lk, 0)
            plsc.store_scatter(buf, [lk, lane + g * L], vec, mask=in_win)

    def process(buf, sem, u, prev):
        @pl.when(prev >= 0)
        def _():
            pltpu.make_async_copy(buf, out_hbm.at[0, pl.ds(0, KCH), :], sem).wait()
            scatter(buf, prev, zeros)

        scatter(buf, u, ones)
        r = u // NKC
        k0 = (u % NKC) * KCH
        pltpu.make_async_copy(buf, out_hbm.at[r, pl.ds(k0, KCH), :], sem).start()

    def pair_step(p, carry):
        prev0, prev1 = carry
        u_a = u0 + 2 * p
        process(buf0, sem0, u_a, prev0)
        u_b = u_a + 1
        valid = u_b < u1

        @pl.when(valid)
        def _():
            process(buf1, sem1, u_b, prev1)

        return u_a, jnp.where(valid, u_b, prev1)

    lax.fori_loop(0, (n + 1) // 2, pair_step, (jnp.int32(-1), jnp.int32(-1)))

    pltpu.make_async_copy(buf0, out_hbm.at[0, pl.ds(0, KCH), :], sem0).wait()

    @pl.when(n >= 2)
    def _():
        pltpu.make_async_copy(buf1, out_hbm.at[0, pl.ds(0, KCH), :], sem1).wait()


@jax.jit
def _one_hot_phys(idx_t):
    mesh = plsc.VectorSubcoreMesh(core_axis_name="c", subcore_axis_name="s")
    run = pl.kernel(
        _body,
        out_type=jax.ShapeDtypeStruct((ROWS, NUM_CLASSES, BATCH), jnp.float32),
        mesh=mesh,
        compiler_params=pltpu.CompilerParams(
            needs_layout_passes=False, use_tc_tiling_on_sc=True
        ),
        scratch_types=[
            pltpu.VMEM((KCH, BATCH), jnp.float32),
            pltpu.VMEM((KCH, BATCH), jnp.float32),
            pltpu.VMEM((NR_PRE * BATCH,), jnp.int32),
            pltpu.SemaphoreType.DMA,
            pltpu.SemaphoreType.DMA,
        ],
    )
    return run(idx_t)


def kernel(inputs):
    idx_t = jnp.transpose(inputs).astype(jnp.int32).reshape(-1)
    idx_t = jnp.concatenate(
        [idx_t, jnp.zeros(((NR_PRE - 1) * BATCH,), jnp.int32)]
    )
    out_phys = _one_hot_phys(idx_t)
    return jnp.transpose(out_phys, (2, 0, 1))
